# initial kernel scaffold (unmeasured)
import jax
import jax.numpy as jnp
from jax import lax
from jax.experimental import pallas as pl
from jax.experimental.pallas import tpu as pltpu

N_DEV = 4


def _ring_allreduce(p):
    M, N = p.shape
    MB = M // N_DEV
    NH = N // 2

    def body(p_ref, out_ref, comm, loc, send_sems, recv_sems, credit_sem,
             copy_sem):
        my = lax.axis_index("i")
        left = (my - 1) % N_DEV
        right = (my + 1) % N_DEV

        for half in range(2):
            col0 = half * NH
            for s in range(6):
                g = half * 6 + s
                send_slot = s % 2
                recv_slot = (s + 1) % 2

                if s == 0:
                    chunk = my
                    cp = pltpu.make_async_copy(
                        p_ref.at[pl.ds(chunk * MB, MB), pl.ds(col0, NH)],
                        loc, copy_sem)
                    cp.start()
                    cp.wait()
                    comm[send_slot] = loc[...]
                elif s <= 3:
                    chunk = (my - s) % N_DEV
                    cp = pltpu.make_async_copy(
                        p_ref.at[pl.ds(chunk * MB, MB), pl.ds(col0, NH)],
                        loc, copy_sem)
                    cp.start()
                    cp.wait()
                    comm[send_slot] = comm[send_slot] + loc[...]
                    if s == 3:
                        st = pltpu.make_async_copy(
                            comm.at[send_slot],
                            out_ref.at[pl.ds(chunk * MB, MB),
                                       pl.ds(col0, NH)],
                            copy_sem)
                        st.start()
                        st.wait()
                else:
                    chunk = (my - s + 4) % N_DEV
                    st = pltpu.make_async_copy(
                        comm.at[send_slot],
                        out_ref.at[pl.ds(chunk * MB, MB), pl.ds(col0, NH)],
                        copy_sem)
                    st.start()
                    st.wait()

                if g >= 2:
                    pl.semaphore_wait(credit_sem, 1)

                rdma = pltpu.make_async_remote_copy(
                    src_ref=comm.at[send_slot],
                    dst_ref=comm.at[recv_slot],
                    send_sem=send_sems.at[send_slot],
                    recv_sem=recv_sems.at[recv_slot],
                    device_id=(right,),
                    device_id_type=pl.DeviceIdType.MESH,
                )
                rdma.start()
                rdma.wait()

                if s == 5:
                    chunk = (my - 2) % N_DEV
                    st = pltpu.make_async_copy(
                        comm.at[recv_slot],
                        out_ref.at[pl.ds(chunk * MB, MB), pl.ds(col0, NH)],
                        copy_sem)
                    st.start()
                    st.wait()

                if 1 <= g <= 10:
                    pl.semaphore_signal(
                        credit_sem, inc=1,
                        device_id=(left,),
                        device_id_type=pl.DeviceIdType.MESH)

    return pl.pallas_call(
        body,
        out_shape=jax.ShapeDtypeStruct((M, N), p.dtype),
        in_specs=[pl.BlockSpec(memory_space=pltpu.ANY)],
        out_specs=pl.BlockSpec(memory_space=pltpu.ANY),
        scratch_shapes=[
            pltpu.VMEM((2, MB, NH), p.dtype),
            pltpu.VMEM((MB, NH), p.dtype),
            pltpu.SemaphoreType.DMA((2,)),
            pltpu.SemaphoreType.DMA((2,)),
            pltpu.SemaphoreType.REGULAR,
            pltpu.SemaphoreType.DMA,
        ],
        compiler_params=pltpu.CompilerParams(collective_id=0),
    )(p)


def kernel(x, w_mat):
    p = lax.dot_general(
        x, w_mat, (((1,), (0,)), ((), ())),
        precision=lax.Precision.HIGHEST,
        preferred_element_type=jnp.float32,
    )
    y = _ring_allreduce(p)
    y = jnp.maximum(y, 0.0)
    amax = jnp.max(jnp.abs(y))
    scale = amax / 127.0
    q = jnp.clip(jnp.round(y / scale), -127.0, 127.0)
    return (q * scale).astype(jnp.float32)


# baseline (device time: 2896217 ns/iter reference)
import jax
import jax.numpy as jnp
from jax import lax
from jax.experimental import pallas as pl
from jax.experimental.pallas import tpu as pltpu

N_DEV = 4


def _ring_allreduce(p):
    M, N = p.shape
    MB = M // N_DEV
    NPARTS = 4
    NH = N // NPARTS
    G_LAST = NPARTS * 6 - 1

    def body(p_ref, out_ref, comm, loc, send_sems, recv_sems, credit_sem,
             copy_sem):
        my = lax.axis_index("i")
        left = (my - 1) % N_DEV
        right = (my + 1) % N_DEV

        for part in range(NPARTS):
            col0 = part * NH
            for s in range(6):
                g = part * 6 + s
                send_slot = s % 2
                recv_slot = (s + 1) % 2

                if s == 0:
                    chunk = my
                    cp = pltpu.make_async_copy(
                        p_ref.at[pl.ds(chunk * MB, MB), pl.ds(col0, NH)],
                        loc, copy_sem)
                    cp.start()
                    cp.wait()
                    comm[send_slot] = loc[...]
                elif s <= 3:
                    chunk = (my - s) % N_DEV
                    cp = pltpu.make_async_copy(
                        p_ref.at[pl.ds(chunk * MB, MB), pl.ds(col0, NH)],
                        loc, copy_sem)
                    cp.start()
                    cp.wait()
                    comm[send_slot] = comm[send_slot] + loc[...]
                    if s == 3:
                        st = pltpu.make_async_copy(
                            comm.at[send_slot],
                            out_ref.at[pl.ds(chunk * MB, MB),
                                       pl.ds(col0, NH)],
                            copy_sem)
                        st.start()
                        st.wait()
                else:
                    chunk = (my - s + 4) % N_DEV
                    st = pltpu.make_async_copy(
                        comm.at[send_slot],
                        out_ref.at[pl.ds(chunk * MB, MB), pl.ds(col0, NH)],
                        copy_sem)
                    st.start()
                    st.wait()

                if g >= 2:
                    pl.semaphore_wait(credit_sem, 1)

                rdma = pltpu.make_async_remote_copy(
                    src_ref=comm.at[send_slot],
                    dst_ref=comm.at[recv_slot],
                    send_sem=send_sems.at[send_slot],
                    recv_sem=recv_sems.at[recv_slot],
                    device_id=(right,),
                    device_id_type=pl.DeviceIdType.MESH,
                )
                rdma.start()
                rdma.wait()

                if s == 5:
                    chunk = (my - 2) % N_DEV
                    st = pltpu.make_async_copy(
                        comm.at[recv_slot],
                        out_ref.at[pl.ds(chunk * MB, MB), pl.ds(col0, NH)],
                        copy_sem)
                    st.start()
                    st.wait()

                if 1 <= g <= G_LAST - 1:
                    pl.semaphore_signal(
                        credit_sem, inc=1,
                        device_id=(left,),
                        device_id_type=pl.DeviceIdType.MESH)

    return pl.pallas_call(
        body,
        out_shape=jax.ShapeDtypeStruct((M, N), p.dtype),
        in_specs=[pl.BlockSpec(memory_space=pl.ANY)],
        out_specs=pl.BlockSpec(memory_space=pl.ANY),
        scratch_shapes=[
            pltpu.VMEM((2, MB, NH), p.dtype),
            pltpu.VMEM((MB, NH), p.dtype),
            pltpu.SemaphoreType.DMA((2,)),
            pltpu.SemaphoreType.DMA((2,)),
            pltpu.SemaphoreType.REGULAR,
            pltpu.SemaphoreType.DMA,
        ],
    )(p)


def kernel(x, w_mat):
    p = lax.dot_general(
        x, w_mat, (((1,), (0,)), ((), ())),
        precision=lax.Precision.HIGHEST,
        preferred_element_type=jnp.float32,
    )
    y = _ring_allreduce(p)
    y = jnp.maximum(y, 0.0)
    amax = jnp.max(jnp.abs(y))
    scale = amax / 127.0
    q = jnp.clip(jnp.round(y / scale), -127.0, 127.0)
    return (q * scale).astype(jnp.float32)


# device time: 1828342 ns/iter; 1.5841x vs baseline; 1.5841x over previous
import jax
import jax.numpy as jnp
from jax import lax
from jax.experimental import pallas as pl
from jax.experimental.pallas import tpu as pltpu

N_DEV = 4


def _ring_allreduce(p):
    M, N = p.shape
    MB = M // N_DEV
    NH = N // 4
    G_LAST = 2 * 6 - 1

    def body(p_ref, out_ref,
             comm_r, loc_r, send_sems_r, recv_sems_r, credit_r,
             comm_l, loc_l, send_sems_l, recv_sems_l, credit_l,
             copy_sem):
        my = lax.axis_index("i")
        left = (my - 1) % N_DEV
        right = (my + 1) % N_DEV

        dirs = (
            (right, left, comm_r, loc_r, send_sems_r, recv_sems_r,
             credit_r, 1, 0),
            (left, right, comm_l, loc_l, send_sems_l, recv_sems_l,
             credit_l, -1, 2),
        )

        for pair in range(2):
            for s in range(6):
                g = pair * 6 + s
                send_slot = s % 2
                recv_slot = (s + 1) % 2

                rdmas = []
                for (dst, peer, comm, loc, ssems, rsems, credit, sign,
                     col_base) in dirs:
                    col0 = (col_base + pair) * NH

                    if s == 0:
                        chunk = my
                        cp = pltpu.make_async_copy(
                            p_ref.at[pl.ds(chunk * MB, MB), pl.ds(col0, NH)],
                            loc, copy_sem)
                        cp.start()
                        cp.wait()
                        comm[send_slot] = loc[...]
                    elif s <= 3:
                        chunk = (my - sign * s) % N_DEV
                        cp = pltpu.make_async_copy(
                            p_ref.at[pl.ds(chunk * MB, MB), pl.ds(col0, NH)],
                            loc, copy_sem)
                        cp.start()
                        cp.wait()
                        comm[send_slot] = comm[send_slot] + loc[...]
                        if s == 3:
                            st = pltpu.make_async_copy(
                                comm.at[send_slot],
                                out_ref.at[pl.ds(chunk * MB, MB),
                                           pl.ds(col0, NH)],
                                copy_sem)
                            st.start()
                            st.wait()
                    else:
                        chunk = (my - sign * s) % N_DEV
                        st = pltpu.make_async_copy(
                            comm.at[send_slot],
                            out_ref.at[pl.ds(chunk * MB, MB),
                                       pl.ds(col0, NH)],
                            copy_sem)
                        st.start()
                        st.wait()

                    if g >= 2:
                        pl.semaphore_wait(credit, 1)

                    rdma = pltpu.make_async_remote_copy(
                        src_ref=comm.at[send_slot],
                        dst_ref=comm.at[recv_slot],
                        send_sem=ssems.at[send_slot],
                        recv_sem=rsems.at[recv_slot],
                        device_id=(dst,),
                        device_id_type=pl.DeviceIdType.MESH,
                    )
                    rdma.start()
                    rdmas.append(rdma)

                for rdma in rdmas:
                    rdma.wait()

                for (dst, peer, comm, loc, ssems, rsems, credit, sign,
                     col_base) in dirs:
                    col0 = (col_base + pair) * NH
                    if s == 5:
                        chunk = (my - sign * 2) % N_DEV
                        st = pltpu.make_async_copy(
                            comm.at[recv_slot],
                            out_ref.at[pl.ds(chunk * MB, MB),
                                       pl.ds(col0, NH)],
                            copy_sem)
                        st.start()
                        st.wait()
                    if 1 <= g <= G_LAST - 1:
                        pl.semaphore_signal(
                            credit, inc=1,
                            device_id=(peer,),
                            device_id_type=pl.DeviceIdType.MESH)

    return pl.pallas_call(
        body,
        out_shape=jax.ShapeDtypeStruct((M, N), p.dtype),
        in_specs=[pl.BlockSpec(memory_space=pl.ANY)],
        out_specs=pl.BlockSpec(memory_space=pl.ANY),
        scratch_shapes=[
            pltpu.VMEM((2, MB, NH), p.dtype),
            pltpu.VMEM((MB, NH), p.dtype),
            pltpu.SemaphoreType.DMA((2,)),
            pltpu.SemaphoreType.DMA((2,)),
            pltpu.SemaphoreType.REGULAR,
            pltpu.VMEM((2, MB, NH), p.dtype),
            pltpu.VMEM((MB, NH), p.dtype),
            pltpu.SemaphoreType.DMA((2,)),
            pltpu.SemaphoreType.DMA((2,)),
            pltpu.SemaphoreType.REGULAR,
            pltpu.SemaphoreType.DMA,
        ],
        compiler_params=pltpu.CompilerParams(
            vmem_limit_bytes=100 * 1024 * 1024,
        ),
    )(p)


def kernel(x, w_mat):
    p = lax.dot_general(
        x, w_mat, (((1,), (0,)), ((), ())),
        precision=lax.Precision.HIGHEST,
        preferred_element_type=jnp.float32,
    )
    y = _ring_allreduce(p)
    y = jnp.maximum(y, 0.0)
    amax = jnp.max(jnp.abs(y))
    scale = amax / 127.0
    q = jnp.clip(jnp.round(y / scale), -127.0, 127.0)
    return (q * scale).astype(jnp.float32)


# device time: 1263998 ns/iter; 2.2913x vs baseline; 1.4465x over previous
import jax
import jax.numpy as jnp
from jax import lax
from jax.experimental import pallas as pl
from jax.experimental.pallas import tpu as pltpu

N_DEV = 4
N_ROUNDS = 4


def _fused_gemm_allreduce(x, w_mat):
    M, K = x.shape
    N = w_mat.shape[1]
    MB = M // N_DEV
    NH = N // (2 * N_ROUNDS)
    G_LAST = N_ROUNDS * 6 - 1

    def body(x_ref, w_ref, out_ref, amax_ref,
             comm_r, blk_r, wbuf_r, xbuf_r, ssems_r, rsems_r, credit_r,
             copy_r,
             comm_l, blk_l, wbuf_l, xbuf_l, ssems_l, rsems_l, credit_l,
             copy_l):
        my = lax.axis_index("i")
        left = (my - 1) % N_DEV
        right = (my + 1) % N_DEV

        dirs = [
            dict(dst=right, up=left, comm=comm_r, blk=blk_r, wbuf=wbuf_r,
                 xbuf=xbuf_r, ssems=ssems_r, rsems=rsems_r,
                 credit=credit_r, copy=copy_r, sign=1, strip0=0),
            dict(dst=left, up=right, comm=comm_l, blk=blk_l, wbuf=wbuf_l,
                 xbuf=xbuf_l, ssems=ssems_l, rsems=rsems_l,
                 credit=credit_l, copy=copy_l, sign=-1, strip0=N_ROUNDS),
        ]

        def load_xchunk(d, chunk):
            cp = pltpu.make_async_copy(
                x_ref.at[pl.ds(chunk * MB, MB), :], d["xbuf"], d["copy"])
            cp.start()
            cp.wait()

        def load_wstrip(d, strip):
            cp = pltpu.make_async_copy(
                w_ref.at[:, pl.ds(strip * NH, NH)], d["wbuf"], d["copy"])
            cp.start()
            cp.wait()

        def gemm_block(d):
            return lax.dot_general(
                d["xbuf"][...], d["wbuf"][...], (((1,), (0,)), ((), ())),
                preferred_element_type=jnp.float32,
            )

        def store_chunk(d, slot, chunk, col0):
            st = pltpu.make_async_copy(
                d["comm"].at[slot],
                out_ref.at[pl.ds(chunk * MB, MB), pl.ds(col0, NH)],
                d["copy"])
            st.start()
            st.wait()

        amax = jnp.float32(0.0)

        for rnd in range(N_ROUNDS):
            for s in range(6):
                g = rnd * 6 + s
                send_slot = s % 2
                recv_slot = (s + 1) % 2

                for d in dirs:
                    if g >= 1:
                        d["prev"].wait_send()
                        if g <= G_LAST - 1:
                            pl.semaphore_signal(
                                d["credit"], inc=1,
                                device_id=(d["up"],),
                                device_id_type=pl.DeviceIdType.MESH)

                if s == 0:
                    if rnd == 0:
                        for d in dirs:
                            load_wstrip(d, d["strip0"])
                            load_xchunk(d, my)
                            d["comm"][0] = gemm_block(d)
                    else:
                        for d in dirs:
                            d["comm"][0] = d["blk"][...]
                else:
                    for d in dirs:
                        d["prev"].wait_recv()
                        if s <= 3:
                            acc = d["comm"][send_slot] + d["blk"][...]
                            if s == 3:
                                acc = jnp.maximum(acc, 0.0)
                            d["comm"][send_slot] = acc

                for d in dirs:
                    if g >= 2:
                        pl.semaphore_wait(d["credit"], 1)
                    rdma = pltpu.make_async_remote_copy(
                        src_ref=d["comm"].at[send_slot],
                        dst_ref=d["comm"].at[recv_slot],
                        send_sem=d["ssems"].at[send_slot],
                        recv_sem=d["rsems"].at[recv_slot],
                        device_id=(d["dst"],),
                        device_id_type=pl.DeviceIdType.MESH,
                    )
                    rdma.start()
                    d["prev"] = rdma

                for d in dirs:
                    col0 = (d["strip0"] + rnd) * NH
                    if s <= 2:
                        nxt = (my - d["sign"] * (s + 1)) % N_DEV
                        load_xchunk(d, nxt)
                        d["blk"][...] = gemm_block(d)
                    if rnd < N_ROUNDS - 1 and s == 4:
                        load_wstrip(d, d["strip0"] + rnd + 1)
                        load_xchunk(d, my)
                        d["blk"][...] = gemm_block(d)
                    if s >= 3:
                        chunk = (my - d["sign"] * s) % N_DEV
                        store_chunk(d, send_slot, chunk, col0)
                        amax = jnp.maximum(amax,
                                           jnp.max(d["comm"][send_slot]))

            for d in dirs:
                col0 = (d["strip0"] + rnd) * NH
                d["prev"].wait_recv()
                chunk = (my - d["sign"] * 2) % N_DEV
                store_chunk(d, 0, chunk, col0)
                amax = jnp.maximum(amax, jnp.max(d["comm"][0]))

        for d in dirs:
            d["prev"].wait_send()
        amax_ref[0, 0] = amax

    return pl.pallas_call(
        body,
        out_shape=(
            jax.ShapeDtypeStruct((M, N), jnp.float32),
            jax.ShapeDtypeStruct((1, 1), jnp.float32),
        ),
        in_specs=[
            pl.BlockSpec(memory_space=pltpu.MemorySpace.VMEM),
            pl.BlockSpec(memory_space=pl.ANY),
        ],
        out_specs=(
            pl.BlockSpec(memory_space=pl.ANY),
            pl.BlockSpec(memory_space=pltpu.MemorySpace.SMEM),
        ),
        scratch_shapes=[
            pltpu.VMEM((2, MB, NH), jnp.float32),
            pltpu.VMEM((MB, NH), jnp.float32),
            pltpu.VMEM((K, NH), jnp.float32),
            pltpu.VMEM((MB, K), jnp.float32),
            pltpu.SemaphoreType.DMA((2,)),
            pltpu.SemaphoreType.DMA((2,)),
            pltpu.SemaphoreType.REGULAR,
            pltpu.SemaphoreType.DMA,
            pltpu.VMEM((2, MB, NH), jnp.float32),
            pltpu.VMEM((MB, NH), jnp.float32),
            pltpu.VMEM((K, NH), jnp.float32),
            pltpu.VMEM((MB, K), jnp.float32),
            pltpu.SemaphoreType.DMA((2,)),
            pltpu.SemaphoreType.DMA((2,)),
            pltpu.SemaphoreType.REGULAR,
            pltpu.SemaphoreType.DMA,
        ],
        compiler_params=pltpu.CompilerParams(
            vmem_limit_bytes=100 * 1024 * 1024,
        ),
    )(x, w_mat)


def kernel(x, w_mat):
    y, amax = _fused_gemm_allreduce(x, w_mat)
    scale = amax[0, 0] / 127.0
    q = jnp.clip(jnp.round(y / scale), -127.0, 127.0)
    return (q * scale).astype(jnp.float32)


# device time: 916889 ns/iter; 3.1587x vs baseline; 1.3786x over previous
import jax
import jax.numpy as jnp
from jax import lax
from jax.experimental import pallas as pl
from jax.experimental.pallas import tpu as pltpu

N_DEV = 4
N_ROUNDS = 4


def _fused_kernel(x, w_mat):
    M, K = x.shape
    N = w_mat.shape[1]
    MB = M // N_DEV
    NH = N // (2 * N_ROUNDS)
    G_LAST = N_ROUNDS * 3 - 1

    def body(x_ref, w_ref, out_ref,
             comm_r, blk_r, wbuf_r, xbuf_r, qcom_r,
             ssems_r, rsems_r, credit_r, qss_r, qrs_r, qcredit_r, copy_r,
             comm_l, blk_l, wbuf_l, xbuf_l, qcom_l,
             ssems_l, rsems_l, credit_l, qss_l, qrs_l, qcredit_l, copy_l,
             mbuf, mss, mrs, mcredit):
        my = lax.axis_index("i")
        left = (my - 1) % N_DEV
        right = (my + 1) % N_DEV

        dirs = [
            dict(dst=right, up=left, comm=comm_r, blk=blk_r, wbuf=wbuf_r,
                 xbuf=xbuf_r, qcom=qcom_r, ssems=ssems_r, rsems=rsems_r,
                 credit=credit_r, qss=qss_r, qrs=qrs_r, qcredit=qcredit_r,
                 copy=copy_r, sign=1, strip0=0),
            dict(dst=left, up=right, comm=comm_l, blk=blk_l, wbuf=wbuf_l,
                 xbuf=xbuf_l, qcom=qcom_l, ssems=ssems_l, rsems=rsems_l,
                 credit=credit_l, qss=qss_l, qrs=qrs_l, qcredit=qcredit_l,
                 copy=copy_l, sign=-1, strip0=N_ROUNDS),
        ]

        def load_xchunk(d, chunk):
            cp = pltpu.make_async_copy(
                x_ref.at[pl.ds(chunk * MB, MB), :], d["xbuf"], d["copy"])
            cp.start()
            cp.wait()

        def load_wstrip(d, strip):
            cp = pltpu.make_async_copy(
                w_ref.at[:, pl.ds(strip * NH, NH)], d["wbuf"], d["copy"])
            cp.start()
            cp.wait()

        def gemm_block(d):
            return lax.dot_general(
                d["xbuf"][...], d["wbuf"][...], (((1,), (0,)), ((), ())),
                preferred_element_type=jnp.float32,
            )

        amax = jnp.float32(0.0)
        for rnd in range(N_ROUNDS):
            for s in range(3):
                g = rnd * 3 + s
                send_slot = s % 2
                recv_slot = (s + 1) % 2

                for d in dirs:
                    if g >= 1:
                        d["prev"].wait_send()
                        if g <= G_LAST - 1:
                            pl.semaphore_signal(
                                d["credit"], inc=1,
                                device_id=(d["up"],),
                                device_id_type=pl.DeviceIdType.MESH)

                if s == 0:
                    if rnd == 0:
                        for d in dirs:
                            load_wstrip(d, d["strip0"])
                            load_xchunk(d, my)
                            d["comm"][0] = gemm_block(d)
                    else:
                        for d in dirs:
                            d["comm"][0] = d["blk"][...]
                else:
                    for d in dirs:
                        d["prev"].wait_recv()
                        d["comm"][send_slot] = (
                            d["comm"][send_slot] + d["blk"][...])

                for d in dirs:
                    if g >= 2:
                        pl.semaphore_wait(d["credit"], 1)
                    rdma = pltpu.make_async_remote_copy(
                        src_ref=d["comm"].at[send_slot],
                        dst_ref=d["comm"].at[recv_slot],
                        send_sem=d["ssems"].at[send_slot],
                        recv_sem=d["rsems"].at[recv_slot],
                        device_id=(d["dst"],),
                        device_id_type=pl.DeviceIdType.MESH,
                    )
                    rdma.start()
                    d["prev"] = rdma

                for d in dirs:
                    nxt = (my - d["sign"] * (s + 1)) % N_DEV
                    load_xchunk(d, nxt)
                    d["blk"][...] = gemm_block(d)
                    if s == 2 and rnd < N_ROUNDS - 1:
                        load_wstrip(d, d["strip0"] + rnd + 1)

            for d in dirs:
                col0 = (d["strip0"] + rnd) * NH
                d["prev"].wait_recv()
                own = (my + d["sign"]) % N_DEV
                d["comm"][1] = jnp.maximum(
                    d["comm"][1] + d["blk"][...], 0.0)
                amax = jnp.maximum(amax, jnp.max(d["comm"][1]))
                st = pltpu.make_async_copy(
                    d["comm"].at[1],
                    out_ref.at[pl.ds(own * MB, MB), pl.ds(col0, NH)],
                    d["copy"])
                st.start()
                st.wait()
                if rnd < N_ROUNDS - 1:
                    load_xchunk(d, my)
                    d["blk"][...] = gemm_block(d)
        for d in dirs:
            d["prev"].wait_send()

        prev = None
        for h in range(3):
            if h >= 1:
                prev.wait_send()
                if h == 1:
                    pl.semaphore_signal(
                        mcredit, inc=1, device_id=(left,),
                        device_id_type=pl.DeviceIdType.MESH)
                prev.wait_recv()
                amax = jnp.maximum(amax, jnp.max(mbuf[h % 2]))
            mbuf[h % 2] = jnp.full((8, 128), amax, jnp.float32)
            if h == 2:
                pl.semaphore_wait(mcredit, 1)
            rdma = pltpu.make_async_remote_copy(
                src_ref=mbuf.at[h % 2],
                dst_ref=mbuf.at[(h + 1) % 2],
                send_sem=mss.at[h % 2],
                recv_sem=mrs.at[(h + 1) % 2],
                device_id=(right,),
                device_id_type=pl.DeviceIdType.MESH,
            )
            rdma.start()
            prev = rdma
        prev.wait_send()
        prev.wait_recv()
        amax = jnp.maximum(amax, jnp.max(mbuf[1]))
        scale = amax / 127.0
        inv_scale = 127.0 / amax

        def dequant_store(d, slot, strip_i, chunk):
            q = d["qcom"][slot, strip_i].astype(jnp.float32) * scale
            d["blk"][...] = q
            st = pltpu.make_async_copy(
                d["blk"],
                out_ref.at[pl.ds(chunk * MB, MB),
                           pl.ds((d["strip0"] + strip_i) * NH, NH)],
                d["copy"])
            st.start()
            st.wait()

        for d in dirs:
            own = (my + d["sign"]) % N_DEV
            for strip_i in range(N_ROUNDS):
                cp = pltpu.make_async_copy(
                    out_ref.at[pl.ds(own * MB, MB),
                               pl.ds((d["strip0"] + strip_i) * NH, NH)],
                    d["blk"], d["copy"])
                cp.start()
                cp.wait()
                q = jnp.clip(jnp.round(d["blk"][...] * inv_scale),
                             -127.0, 127.0)
                d["qcom"][0, strip_i] = q.astype(jnp.int8)

        for h in range(3):
            send_slot = h % 2
            recv_slot = (h + 1) % 2
            for d in dirs:
                if h >= 1:
                    d["qprev"].wait_send()
                    if h == 1:
                        pl.semaphore_signal(
                            d["qcredit"], inc=1, device_id=(d["up"],),
                            device_id_type=pl.DeviceIdType.MESH)
                    d["qprev"].wait_recv()
            for d in dirs:
                if h == 2:
                    pl.semaphore_wait(d["qcredit"], 1)
                rdma = pltpu.make_async_remote_copy(
                    src_ref=d["qcom"].at[send_slot],
                    dst_ref=d["qcom"].at[recv_slot],
                    send_sem=d["qss"].at[send_slot],
                    recv_sem=d["qrs"].at[recv_slot],
                    device_id=(d["dst"],),
                    device_id_type=pl.DeviceIdType.MESH,
                )
                rdma.start()
                d["qprev"] = rdma
            for d in dirs:
                chunk = (my + d["sign"] * (1 - h)) % N_DEV
                for strip_i in range(N_ROUNDS):
                    dequant_store(d, send_slot, strip_i, chunk)
        for d in dirs:
            d["qprev"].wait_send()
            d["qprev"].wait_recv()
            chunk = (my - d["sign"] * 2) % N_DEV
            for strip_i in range(N_ROUNDS):
                dequant_store(d, 1, strip_i, chunk)

    return pl.pallas_call(
        body,
        out_shape=jax.ShapeDtypeStruct((M, N), jnp.float32),
        in_specs=[
            pl.BlockSpec(memory_space=pl.ANY),
            pl.BlockSpec(memory_space=pl.ANY),
        ],
        out_specs=pl.BlockSpec(memory_space=pl.ANY),
        scratch_shapes=[
            pltpu.VMEM((2, MB, NH), jnp.float32),
            pltpu.VMEM((MB, NH), jnp.float32),
            pltpu.VMEM((K, NH), jnp.float32),
            pltpu.VMEM((MB, K), jnp.float32),
            pltpu.VMEM((2, N_ROUNDS, MB, NH), jnp.int8),
            pltpu.SemaphoreType.DMA((2,)),
            pltpu.SemaphoreType.DMA((2,)),
            pltpu.SemaphoreType.REGULAR,
            pltpu.SemaphoreType.DMA((2,)),
            pltpu.SemaphoreType.DMA((2,)),
            pltpu.SemaphoreType.REGULAR,
            pltpu.SemaphoreType.DMA,
            pltpu.VMEM((2, MB, NH), jnp.float32),
            pltpu.VMEM((MB, NH), jnp.float32),
            pltpu.VMEM((K, NH), jnp.float32),
            pltpu.VMEM((MB, K), jnp.float32),
            pltpu.VMEM((2, N_ROUNDS, MB, NH), jnp.int8),
            pltpu.SemaphoreType.DMA((2,)),
            pltpu.SemaphoreType.DMA((2,)),
            pltpu.SemaphoreType.REGULAR,
            pltpu.SemaphoreType.DMA((2,)),
            pltpu.SemaphoreType.DMA((2,)),
            pltpu.SemaphoreType.REGULAR,
            pltpu.SemaphoreType.DMA,
            pltpu.VMEM((2, 8, 128), jnp.float32),
            pltpu.SemaphoreType.DMA((2,)),
            pltpu.SemaphoreType.DMA((2,)),
            pltpu.SemaphoreType.REGULAR,
        ],
        compiler_params=pltpu.CompilerParams(
            vmem_limit_bytes=100 * 1024 * 1024,
        ),
    )(x, w_mat)


def kernel(x, w_mat):
    return _fused_kernel(x, w_mat)


# device time: 645705 ns/iter; 4.4854x vs baseline; 1.4200x over previous
import jax
import jax.numpy as jnp
from jax import lax
from jax.experimental import pallas as pl
from jax.experimental.pallas import tpu as pltpu

N_DEV = 4
N_ROUNDS = 4


def _fused_kernel(x, w_mat):
    M, K = x.shape
    N = w_mat.shape[1]
    MB = M // N_DEV
    NH = N // (2 * N_ROUNDS)
    G_LAST = N_ROUNDS * 3 - 1

    def body(x_ref, w_ref, out_ref,
             comm_r, blk_r, wbuf_r, xbuf_r, qcom_r,
             ssems_r, rsems_r, credit_r, qss_r, qrs_r, qcredit_r, copy_r,
             comm_l, blk_l, wbuf_l, xbuf_l, qcom_l,
             ssems_l, rsems_l, credit_l, qss_l, qrs_l, qcredit_l, copy_l,
             mbuf, mss, mrs, mcredit):
        my = lax.axis_index("i")
        left = (my - 1) % N_DEV
        right = (my + 1) % N_DEV

        dirs = [
            dict(dst=right, up=left, comm=comm_r, blk=blk_r, wbuf=wbuf_r,
                 xbuf=xbuf_r, qcom=qcom_r, ssems=ssems_r, rsems=rsems_r,
                 credit=credit_r, qss=qss_r, qrs=qrs_r, qcredit=qcredit_r,
                 copy=copy_r, sign=1, strip0=0),
            dict(dst=left, up=right, comm=comm_l, blk=blk_l, wbuf=wbuf_l,
                 xbuf=xbuf_l, qcom=qcom_l, ssems=ssems_l, rsems=rsems_l,
                 credit=credit_l, qss=qss_l, qrs=qrs_l, qcredit=qcredit_l,
                 copy=copy_l, sign=-1, strip0=N_ROUNDS),
        ]

        def load_xchunk(d, chunk):
            cp = pltpu.make_async_copy(
                x_ref.at[pl.ds(chunk * MB, MB), :], d["xbuf"], d["copy"])
            cp.start()
            cp.wait()

        def load_wstrip(d, strip):
            cp = pltpu.make_async_copy(
                w_ref.at[:, pl.ds(strip * NH, NH)], d["wbuf"], d["copy"])
            cp.start()
            cp.wait()

        def gemm_block(d):
            return lax.dot_general(
                d["xbuf"][...], d["wbuf"][...], (((1,), (0,)), ((), ())),
                preferred_element_type=jnp.float32,
            )

        amax = jnp.float32(0.0)
        for rnd in range(N_ROUNDS):
            for s in range(3):
                g = rnd * 3 + s
                send_slot = s % 2
                recv_slot = (s + 1) % 2

                for d in dirs:
                    if g >= 1:
                        d["prev"].wait_send()
                        if g <= G_LAST - 1:
                            pl.semaphore_signal(
                                d["credit"], inc=1,
                                device_id=(d["up"],),
                                device_id_type=pl.DeviceIdType.MESH)

                if s == 0:
                    if rnd == 0:
                        for d in dirs:
                            load_wstrip(d, d["strip0"])
                            load_xchunk(d, my)
                            d["comm"][0] = gemm_block(d).astype(
                                jnp.bfloat16)
                    else:
                        for d in dirs:
                            d["comm"][0] = d["blk"][...].astype(
                                jnp.bfloat16)
                else:
                    for d in dirs:
                        d["prev"].wait_recv()
                        d["comm"][send_slot] = (
                            d["comm"][send_slot].astype(jnp.float32)
                            + d["blk"][...]).astype(jnp.bfloat16)

                for d in dirs:
                    if g >= 2:
                        pl.semaphore_wait(d["credit"], 1)
                    rdma = pltpu.make_async_remote_copy(
                        src_ref=d["comm"].at[send_slot],
                        dst_ref=d["comm"].at[recv_slot],
                        send_sem=d["ssems"].at[send_slot],
                        recv_sem=d["rsems"].at[recv_slot],
                        device_id=(d["dst"],),
                        device_id_type=pl.DeviceIdType.MESH,
                    )
                    rdma.start()
                    d["prev"] = rdma

                for d in dirs:
                    nxt = (my - d["sign"] * (s + 1)) % N_DEV
                    load_xchunk(d, nxt)
                    d["blk"][...] = gemm_block(d)
                    if s == 2 and rnd < N_ROUNDS - 1:
                        load_wstrip(d, d["strip0"] + rnd + 1)

            for d in dirs:
                col0 = (d["strip0"] + rnd) * NH
                d["prev"].wait_recv()
                own = (my + d["sign"]) % N_DEV
                d["blk"][...] = jnp.maximum(
                    d["comm"][1].astype(jnp.float32) + d["blk"][...], 0.0)
                amax = jnp.maximum(amax, jnp.max(d["blk"][...]))
                st = pltpu.make_async_copy(
                    d["blk"],
                    out_ref.at[pl.ds(own * MB, MB), pl.ds(col0, NH)],
                    d["copy"])
                st.start()
                st.wait()
                if rnd < N_ROUNDS - 1:
                    load_xchunk(d, my)
                    d["blk"][...] = gemm_block(d)
        for d in dirs:
            d["prev"].wait_send()

        prev = None
        for h in range(3):
            if h >= 1:
                prev.wait_send()
                if h == 1:
                    pl.semaphore_signal(
                        mcredit, inc=1, device_id=(left,),
                        device_id_type=pl.DeviceIdType.MESH)
                prev.wait_recv()
                amax = jnp.maximum(amax, jnp.max(mbuf[h % 2]))
            mbuf[h % 2] = jnp.full((8, 128), amax, jnp.float32)
            if h == 2:
                pl.semaphore_wait(mcredit, 1)
            rdma = pltpu.make_async_remote_copy(
                src_ref=mbuf.at[h % 2],
                dst_ref=mbuf.at[(h + 1) % 2],
                send_sem=mss.at[h % 2],
                recv_sem=mrs.at[(h + 1) % 2],
                device_id=(right,),
                device_id_type=pl.DeviceIdType.MESH,
            )
            rdma.start()
            prev = rdma
        prev.wait_send()
        prev.wait_recv()
        amax = jnp.maximum(amax, jnp.max(mbuf[1]))
        scale = amax / 127.0
        inv_scale = 127.0 / amax

        def dequant_store(d, slot, strip_i, chunk):
            q = d["qcom"][slot, strip_i].astype(jnp.float32) * scale
            d["blk"][...] = q
            st = pltpu.make_async_copy(
                d["blk"],
                out_ref.at[pl.ds(chunk * MB, MB),
                           pl.ds((d["strip0"] + strip_i) * NH, NH)],
                d["copy"])
            st.start()
            st.wait()

        for d in dirs:
            own = (my + d["sign"]) % N_DEV
            for strip_i in range(N_ROUNDS):
                cp = pltpu.make_async_copy(
                    out_ref.at[pl.ds(own * MB, MB),
                               pl.ds((d["strip0"] + strip_i) * NH, NH)],
                    d["blk"], d["copy"])
                cp.start()
                cp.wait()
                q = jnp.clip(jnp.round(d["blk"][...] * inv_scale),
                             -127.0, 127.0)
                d["qcom"][0, strip_i] = q.astype(jnp.int8)

        for h in range(3):
            send_slot = h % 2
            recv_slot = (h + 1) % 2
            for d in dirs:
                if h >= 1:
                    d["qprev"].wait_send()
                    if h == 1:
                        pl.semaphore_signal(
                            d["qcredit"], inc=1, device_id=(d["up"],),
                            device_id_type=pl.DeviceIdType.MESH)
                    d["qprev"].wait_recv()
            for d in dirs:
                if h == 2:
                    pl.semaphore_wait(d["qcredit"], 1)
                rdma = pltpu.make_async_remote_copy(
                    src_ref=d["qcom"].at[send_slot],
                    dst_ref=d["qcom"].at[recv_slot],
                    send_sem=d["qss"].at[send_slot],
                    recv_sem=d["qrs"].at[recv_slot],
                    device_id=(d["dst"],),
                    device_id_type=pl.DeviceIdType.MESH,
                )
                rdma.start()
                d["qprev"] = rdma
            for d in dirs:
                chunk = (my + d["sign"] * (1 - h)) % N_DEV
                for strip_i in range(N_ROUNDS):
                    dequant_store(d, send_slot, strip_i, chunk)
        for d in dirs:
            d["qprev"].wait_send()
            d["qprev"].wait_recv()
            chunk = (my - d["sign"] * 2) % N_DEV
            for strip_i in range(N_ROUNDS):
                dequant_store(d, 1, strip_i, chunk)

    return pl.pallas_call(
        body,
        out_shape=jax.ShapeDtypeStruct((M, N), jnp.float32),
        in_specs=[
            pl.BlockSpec(memory_space=pl.ANY),
            pl.BlockSpec(memory_space=pl.ANY),
        ],
        out_specs=pl.BlockSpec(memory_space=pl.ANY),
        scratch_shapes=[
            pltpu.VMEM((2, MB, NH), jnp.bfloat16),
            pltpu.VMEM((MB, NH), jnp.float32),
            pltpu.VMEM((K, NH), jnp.float32),
            pltpu.VMEM((MB, K), jnp.float32),
            pltpu.VMEM((2, N_ROUNDS, MB, NH), jnp.int8),
            pltpu.SemaphoreType.DMA((2,)),
            pltpu.SemaphoreType.DMA((2,)),
            pltpu.SemaphoreType.REGULAR,
            pltpu.SemaphoreType.DMA((2,)),
            pltpu.SemaphoreType.DMA((2,)),
            pltpu.SemaphoreType.REGULAR,
            pltpu.SemaphoreType.DMA,
            pltpu.VMEM((2, MB, NH), jnp.bfloat16),
            pltpu.VMEM((MB, NH), jnp.float32),
            pltpu.VMEM((K, NH), jnp.float32),
            pltpu.VMEM((MB, K), jnp.float32),
            pltpu.VMEM((2, N_ROUNDS, MB, NH), jnp.int8),
            pltpu.SemaphoreType.DMA((2,)),
            pltpu.SemaphoreType.DMA((2,)),
            pltpu.SemaphoreType.REGULAR,
            pltpu.SemaphoreType.DMA((2,)),
            pltpu.SemaphoreType.DMA((2,)),
            pltpu.SemaphoreType.REGULAR,
            pltpu.SemaphoreType.DMA,
            pltpu.VMEM((2, 8, 128), jnp.float32),
            pltpu.SemaphoreType.DMA((2,)),
            pltpu.SemaphoreType.DMA((2,)),
            pltpu.SemaphoreType.REGULAR,
        ],
        compiler_params=pltpu.CompilerParams(
            vmem_limit_bytes=100 * 1024 * 1024,
        ),
    )(x, w_mat)


def kernel(x, w_mat):
    return _fused_kernel(x, w_mat)
